# baseline (speedup 1.0000x reference)
import functools

import jax
import jax.numpy as jnp
from jax import lax
from jax.experimental import pallas as pl
from jax.experimental.pallas import tpu as pltpu
from jax.experimental.pallas import tpu_sc as plsc

N = 10000
E = 160000
NW = 32
B = 128
NB = 40
E_PAD = NW * NB * B
N_ACC = 10240
RPS = N_ACC // 16

_MESH = plsc.VectorSubcoreMesh(core_axis_name="c", subcore_axis_name="s")


def _make_agg(nchunks, with_counts):
    nouts = nchunks + (1 if with_counts else 0)
    scratch = [
        pltpu.VMEM((NB, B), jnp.int32),
        pltpu.VMEM((NB, B), jnp.int32),
        pltpu.VMEM((B, 128), jnp.float32),
        pltpu.VMEM((B, 128), jnp.float32),
        pltpu.VMEM_SHARED((N_ACC, 128), jnp.float32),
        pltpu.SemaphoreType.DMA,
        pltpu.SemaphoreType.DMA,
    ]

    @functools.partial(
        pl.kernel, mesh=_MESH,
        out_type=jax.ShapeDtypeStruct((nouts, 2, N_ACC, 128), jnp.float32),
        scratch_types=scratch,
    )
    def agg(*refs):
        tables = refs[:nchunks]
        src_hbm, dst_hbm, zeros_hbm = refs[nchunks:nchunks + 3]
        out_hbm = refs[nchunks + 3]
        src_v, dst_v, rows_a, rows_b, acc_sh, sem_a, sem_b = \
            refs[nchunks + 4:]

        c = lax.axis_index("c")
        s = lax.axis_index("s")
        wid = s * 2 + c
        pltpu.sync_copy(zeros_hbm, acc_sh.at[pl.ds(s * RPS, RPS)])
        pltpu.sync_copy(src_hbm.at[wid], src_v)
        pltpu.sync_copy(dst_hbm.at[wid], dst_v)
        plsc.subcore_barrier()

        def flush(k):
            pltpu.sync_copy(acc_sh.at[pl.ds(s * RPS, RPS)],
                            out_hbm.at[k, c, pl.ds(s * RPS, RPS)])
            pltpu.sync_copy(zeros_hbm, acc_sh.at[pl.ds(s * RPS, RPS)])
            plsc.subcore_barrier()

        for k in range(nchunks):
            table = tables[k]
            pltpu.async_copy(table.at[src_v.at[0]], rows_a, sem_a)

            def body(h, carry):
                j0 = 2 * h
                pltpu.async_copy(table.at[src_v.at[j0 + 1]], rows_b, sem_b)
                pltpu.make_async_copy(
                    table.at[src_v.at[0]], rows_a, sem_a).wait()
                pltpu.sync_copy(rows_a, acc_sh.at[dst_v.at[j0]], add=True)

                @pl.when(h < NB // 2 - 1)
                def _():
                    pltpu.async_copy(
                        table.at[src_v.at[j0 + 2]], rows_a, sem_a)

                pltpu.make_async_copy(
                    table.at[src_v.at[0]], rows_b, sem_b).wait()
                pltpu.sync_copy(rows_b, acc_sh.at[dst_v.at[j0 + 1]], add=True)
                return carry

            lax.fori_loop(0, NB // 2, body, 0)
            plsc.subcore_barrier()
            flush(k)

        if with_counts:
            ones16 = jnp.full((16,), 1.0, jnp.float32)

            def obody(i2, carry):
                for k2 in range(8):
                    rows_a[i2, pl.ds(k2 * 16, 16)] = ones16
                return carry

            lax.fori_loop(0, B, obody, 0)

            def cbody(j, carry):
                pltpu.sync_copy(rows_a, acc_sh.at[dst_v.at[j]], add=True)
                return carry

            lax.fori_loop(0, NB, cbody, 0)
            plsc.subcore_barrier()
            flush(nchunks)

    return agg


_agg_l1 = _make_agg(2, with_counts=True)
_agg_l2 = _make_agg(4, with_counts=False)


def _dense(parts, cnt, x, Wl, b, Wr, relu):
    NP, D = x.shape
    F = Wl.shape[1]
    BN = 512

    def body(p_ref, c_ref, x_ref, wl_ref, b_ref, wr_ref, o_ref):
        cnt_b = jnp.maximum(c_ref[0] + c_ref[1], 1.0)
        mean = (p_ref[0] + p_ref[1]) / cnt_b
        acc = jnp.dot(mean, wl_ref[...], preferred_element_type=jnp.float32)
        acc = acc + jnp.dot(x_ref[...], wr_ref[...],
                            preferred_element_type=jnp.float32)
        acc = acc + b_ref[...]
        if relu:
            acc = jnp.maximum(acc, 0.0)
        o_ref[...] = acc

    return pl.pallas_call(
        body,
        grid=(NP // BN,),
        in_specs=[
            pl.BlockSpec((2, BN, D), lambda i: (0, i, 0)),
            pl.BlockSpec((2, BN, 1), lambda i: (0, i, 0)),
            pl.BlockSpec((BN, D), lambda i: (i, 0)),
            pl.BlockSpec((D, F), lambda i: (0, 0)),
            pl.BlockSpec((1, F), lambda i: (0, 0)),
            pl.BlockSpec((D, F), lambda i: (0, 0)),
        ],
        out_specs=pl.BlockSpec((BN, F), lambda i: (i, 0)),
        out_shape=jax.ShapeDtypeStruct((NP, F), jnp.float32),
    )(parts, cnt, x, Wl, b, Wr)


def kernel(x, edge_index, W1l, b1l, W1r, W2l, b2l, W2r):
    src = edge_index[0]
    dst = edge_index[1]
    pad = E_PAD - E
    src3 = jnp.concatenate(
        [src, jnp.zeros((pad,), jnp.int32)]).reshape(NW, NB, B)
    dst3 = jnp.concatenate(
        [dst, jnp.full((pad,), N, jnp.int32)]).reshape(NW, NB, B)
    zeros = jnp.zeros((RPS, 128), jnp.float32)

    agg1 = _agg_l1(x[:, :128], x[:, 128:], src3, dst3, zeros)
    parts1 = jnp.concatenate([agg1[0], agg1[1]], axis=2)
    cnt = agg1[2][:, :, 0:1]

    x_pad = jnp.pad(x, ((0, N_ACC - N), (0, 0)))
    h = _dense(parts1, cnt, x_pad, W1l, b1l.reshape(1, -1), W1r, relu=True)

    q = _agg_l2(h[:, 0:128], h[:, 128:256], h[:, 256:384], h[:, 384:512],
                src3, dst3, zeros)
    parts2 = jnp.concatenate([q[0], q[1], q[2], q[3]], axis=2)
    out = _dense(parts2, cnt, h, W2l, b2l.reshape(1, -1), W2r, relu=False)
    return out[:N]


# speedup vs baseline: 1.0620x; 1.0620x over previous
import functools

import jax
import jax.numpy as jnp
from jax import lax
from jax.experimental import pallas as pl
from jax.experimental.pallas import tpu as pltpu
from jax.experimental.pallas import tpu_sc as plsc

N = 10000
E = 160000
NW = 32
B = 128
NB = 40
E_PAD = NW * NB * B
N_ACC = 10240
RPS = N_ACC // 16

_MESH = plsc.VectorSubcoreMesh(core_axis_name="c", subcore_axis_name="s")


@functools.partial(
    pl.kernel, mesh=_MESH,
    out_type=jax.ShapeDtypeStruct((2, N_ACC, 128), jnp.float32),
    scratch_types=[
        pltpu.VMEM((NB, B), jnp.int32),
        pltpu.VMEM((NB, B), jnp.int32),
        pltpu.VMEM((B, 128), jnp.float32),
        pltpu.VMEM((B, 128), jnp.float32),
        pltpu.VMEM_SHARED((N_ACC, 128), jnp.float32),
        pltpu.SemaphoreType.DMA,
        pltpu.SemaphoreType.DMA,
        pltpu.SemaphoreType.DMA,
        pltpu.SemaphoreType.DMA,
    ],
)
def _agg(table_hbm, src_hbm, dst_hbm, zeros_hbm, out_hbm,
         src_v, dst_v, rows_a, rows_b, acc_sh, sga, sgb, ssa, ssb):
    c = lax.axis_index("c")
    s = lax.axis_index("s")
    wid = s * 2 + c
    pltpu.sync_copy(zeros_hbm, acc_sh.at[pl.ds(s * RPS, RPS)])
    pltpu.sync_copy(src_hbm.at[wid], src_v)
    pltpu.sync_copy(dst_hbm.at[wid], dst_v)
    plsc.subcore_barrier()

    pltpu.async_copy(table_hbm.at[src_v.at[0]], rows_a, sga)
    pltpu.async_copy(table_hbm.at[src_v.at[1]], rows_b, sgb)

    def body(h, carry):
        j0 = 2 * h
        pltpu.make_async_copy(table_hbm.at[src_v.at[0]], rows_a, sga).wait()
        pltpu.async_copy(rows_a, acc_sh.at[dst_v.at[j0]], ssa, add=True)
        pltpu.make_async_copy(table_hbm.at[src_v.at[0]], rows_b, sgb).wait()
        pltpu.async_copy(rows_b, acc_sh.at[dst_v.at[j0 + 1]], ssb, add=True)

        @pl.when(h < NB // 2 - 1)
        def _():
            pltpu.make_async_copy(
                rows_a, acc_sh.at[dst_v.at[0]], ssa).wait()
            pltpu.async_copy(table_hbm.at[src_v.at[j0 + 2]], rows_a, sga)
            pltpu.make_async_copy(
                rows_b, acc_sh.at[dst_v.at[0]], ssb).wait()
            pltpu.async_copy(table_hbm.at[src_v.at[j0 + 3]], rows_b, sgb)

        return carry

    lax.fori_loop(0, NB // 2, body, 0)
    pltpu.make_async_copy(rows_a, acc_sh.at[dst_v.at[0]], ssa).wait()
    pltpu.make_async_copy(rows_b, acc_sh.at[dst_v.at[0]], ssb).wait()
    plsc.subcore_barrier()
    pltpu.sync_copy(acc_sh.at[pl.ds(s * RPS, RPS)],
                    out_hbm.at[c, pl.ds(s * RPS, RPS)])


@functools.partial(
    pl.kernel, mesh=_MESH,
    out_type=jax.ShapeDtypeStruct((2, N_ACC, 128), jnp.float32),
    scratch_types=[
        pltpu.VMEM((NB, B), jnp.int32),
        pltpu.VMEM((B, 128), jnp.float32),
        pltpu.VMEM_SHARED((N_ACC, 128), jnp.float32),
        pltpu.SemaphoreType.DMA,
    ],
)
def _counts(ones_hbm, dst_hbm, zeros_hbm, out_hbm, dst_v, ones_v, acc_sh,
            sem):
    c = lax.axis_index("c")
    s = lax.axis_index("s")
    wid = s * 2 + c
    pltpu.sync_copy(zeros_hbm, acc_sh.at[pl.ds(s * RPS, RPS)])
    pltpu.sync_copy(dst_hbm.at[wid], dst_v)
    pltpu.sync_copy(ones_hbm, ones_v)
    plsc.subcore_barrier()

    def fire(j, carry):
        pltpu.async_copy(ones_v, acc_sh.at[dst_v.at[j]], sem, add=True)
        return carry

    def drain(j, carry):
        pltpu.make_async_copy(ones_v, acc_sh.at[dst_v.at[0]], sem).wait()
        return carry

    lax.fori_loop(0, NB, fire, 0)
    lax.fori_loop(0, NB, drain, 0)
    plsc.subcore_barrier()
    pltpu.sync_copy(acc_sh.at[pl.ds(s * RPS, RPS)],
                    out_hbm.at[c, pl.ds(s * RPS, RPS)])


def _dense(parts, cnt, x, Wl, b, Wr, relu):
    NP, D = x.shape
    F = Wl.shape[1]
    BN = 512

    def body(p_ref, c_ref, x_ref, wl_ref, b_ref, wr_ref, o_ref):
        cnt_b = jnp.maximum(c_ref[0] + c_ref[1], 1.0)
        mean = (p_ref[0] + p_ref[1]) / cnt_b
        acc = jnp.dot(mean, wl_ref[...], preferred_element_type=jnp.float32)
        acc = acc + jnp.dot(x_ref[...], wr_ref[...],
                            preferred_element_type=jnp.float32)
        acc = acc + b_ref[...]
        if relu:
            acc = jnp.maximum(acc, 0.0)
        o_ref[...] = acc

    return pl.pallas_call(
        body,
        grid=(NP // BN,),
        in_specs=[
            pl.BlockSpec((2, BN, D), lambda i: (0, i, 0)),
            pl.BlockSpec((2, BN, 1), lambda i: (0, i, 0)),
            pl.BlockSpec((BN, D), lambda i: (i, 0)),
            pl.BlockSpec((D, F), lambda i: (0, 0)),
            pl.BlockSpec((1, F), lambda i: (0, 0)),
            pl.BlockSpec((D, F), lambda i: (0, 0)),
        ],
        out_specs=pl.BlockSpec((BN, F), lambda i: (i, 0)),
        out_shape=jax.ShapeDtypeStruct((NP, F), jnp.float32),
    )(parts, cnt, x, Wl, b, Wr)


def kernel(x, edge_index, W1l, b1l, W1r, W2l, b2l, W2r):
    src = edge_index[0]
    dst = edge_index[1]
    pad = E_PAD - E
    src3 = jnp.concatenate(
        [src, jnp.zeros((pad,), jnp.int32)]).reshape(NW, NB, B)
    dst3 = jnp.concatenate(
        [dst, jnp.full((pad,), N, jnp.int32)]).reshape(NW, NB, B)
    zeros = jnp.zeros((RPS, 128), jnp.float32)
    ones = jnp.ones((B, 128), jnp.float32)

    cntp = _counts(ones, dst3, zeros)
    cnt = cntp[:, :, 0:1]

    p0 = _agg(x[:, :128], src3, dst3, zeros)
    p1 = _agg(x[:, 128:], src3, dst3, zeros)
    parts1 = jnp.concatenate([p0, p1], axis=2)

    x_pad = jnp.pad(x, ((0, N_ACC - N), (0, 0)))
    h = _dense(parts1, cnt, x_pad, W1l, b1l.reshape(1, -1), W1r, relu=True)

    p2 = [_agg(h[:, k * 128:(k + 1) * 128], src3, dst3, zeros)
          for k in range(4)]
    parts2 = jnp.concatenate(p2, axis=2)
    out = _dense(parts2, cnt, h, W2l, b2l.reshape(1, -1), W2r, relu=False)
    return out[:N]


# speedup vs baseline: 1.0919x; 1.0282x over previous
import functools

import jax
import jax.numpy as jnp
from jax import lax
from jax.experimental import pallas as pl
from jax.experimental.pallas import tpu as pltpu
from jax.experimental.pallas import tpu_sc as plsc

N = 10000
E = 160000
NW = 32
B = 128
NB = 40
E_PAD = NW * NB * B
N_ACC = 10240
RPS = N_ACC // 16

_MESH = plsc.VectorSubcoreMesh(core_axis_name="c", subcore_axis_name="s")


@functools.partial(
    pl.kernel, mesh=_MESH,
    out_type=jax.ShapeDtypeStruct((2, N_ACC, 128), jnp.float32),
    scratch_types=[
        pltpu.VMEM((NB, B), jnp.int32),
        pltpu.VMEM((NB, B), jnp.int32),
        pltpu.VMEM((B, 128), jnp.float32),
        pltpu.VMEM((B, 128), jnp.float32),
        pltpu.VMEM_SHARED((N_ACC, 128), jnp.float32),
        pltpu.SemaphoreType.DMA,
        pltpu.SemaphoreType.DMA,
    ],
)
def _agg(table_hbm, src_hbm, dst_hbm, zeros_hbm, out_hbm,
         src_v, dst_v, rows_a, rows_b, acc_sh, sga, sgb):
    c = lax.axis_index("c")
    s = lax.axis_index("s")
    wid = s * 2 + c
    pltpu.sync_copy(zeros_hbm, acc_sh.at[pl.ds(s * RPS, RPS)])
    pltpu.sync_copy(src_hbm.at[wid], src_v)
    pltpu.sync_copy(dst_hbm.at[wid], dst_v)
    plsc.subcore_barrier()

    pltpu.async_copy(table_hbm.at[src_v.at[0]], rows_a, sga)

    def body(h, carry):
        j0 = 2 * h
        pltpu.async_copy(table_hbm.at[src_v.at[j0 + 1]], rows_b, sgb)
        pltpu.make_async_copy(table_hbm.at[src_v.at[0]], rows_a, sga).wait()
        pltpu.sync_copy(rows_a, acc_sh.at[dst_v.at[j0]], add=True)

        @pl.when(h < NB // 2 - 1)
        def _():
            pltpu.async_copy(table_hbm.at[src_v.at[j0 + 2]], rows_a, sga)

        pltpu.make_async_copy(table_hbm.at[src_v.at[0]], rows_b, sgb).wait()
        pltpu.sync_copy(rows_b, acc_sh.at[dst_v.at[j0 + 1]], add=True)
        return carry

    lax.fori_loop(0, NB // 2, body, 0)
    plsc.subcore_barrier()
    pltpu.sync_copy(acc_sh.at[pl.ds(s * RPS, RPS)],
                    out_hbm.at[c, pl.ds(s * RPS, RPS)])


@functools.partial(
    pl.kernel, mesh=_MESH,
    out_type=jax.ShapeDtypeStruct((2, N_ACC, 128), jnp.float32),
    scratch_types=[
        pltpu.VMEM((NB, B), jnp.int32),
        pltpu.VMEM((B, 128), jnp.float32),
        pltpu.VMEM_SHARED((N_ACC, 128), jnp.float32),
        pltpu.SemaphoreType.DMA,
    ],
)
def _counts(ones_hbm, dst_hbm, zeros_hbm, out_hbm, dst_v, ones_v, acc_sh,
            sem):
    c = lax.axis_index("c")
    s = lax.axis_index("s")
    wid = s * 2 + c
    pltpu.sync_copy(zeros_hbm, acc_sh.at[pl.ds(s * RPS, RPS)])
    pltpu.sync_copy(dst_hbm.at[wid], dst_v)
    pltpu.sync_copy(ones_hbm, ones_v)
    plsc.subcore_barrier()

    def fire(j, carry):
        pltpu.async_copy(ones_v, acc_sh.at[dst_v.at[j]], sem, add=True)
        return carry

    def drain(j, carry):
        pltpu.make_async_copy(ones_v, acc_sh.at[dst_v.at[0]], sem).wait()
        return carry

    lax.fori_loop(0, NB, fire, 0)
    lax.fori_loop(0, NB, drain, 0)
    plsc.subcore_barrier()
    pltpu.sync_copy(acc_sh.at[pl.ds(s * RPS, RPS)],
                    out_hbm.at[c, pl.ds(s * RPS, RPS)])


def _dense(parts, cnt, x, Wl, b, Wr, relu):
    NP, D = x.shape
    F = Wl.shape[1]
    BN = 512

    def body(p_ref, c_ref, x_ref, wl_ref, b_ref, wr_ref, o_ref):
        cnt_b = jnp.maximum(c_ref[0] + c_ref[1], 1.0)
        mean = (p_ref[0] + p_ref[1]) / cnt_b
        acc = jnp.dot(mean, wl_ref[...], preferred_element_type=jnp.float32)
        acc = acc + jnp.dot(x_ref[...], wr_ref[...],
                            preferred_element_type=jnp.float32)
        acc = acc + b_ref[...]
        if relu:
            acc = jnp.maximum(acc, 0.0)
        o_ref[...] = acc

    return pl.pallas_call(
        body,
        grid=(NP // BN,),
        in_specs=[
            pl.BlockSpec((2, BN, D), lambda i: (0, i, 0)),
            pl.BlockSpec((2, BN, 1), lambda i: (0, i, 0)),
            pl.BlockSpec((BN, D), lambda i: (i, 0)),
            pl.BlockSpec((D, F), lambda i: (0, 0)),
            pl.BlockSpec((1, F), lambda i: (0, 0)),
            pl.BlockSpec((D, F), lambda i: (0, 0)),
        ],
        out_specs=pl.BlockSpec((BN, F), lambda i: (i, 0)),
        out_shape=jax.ShapeDtypeStruct((NP, F), jnp.float32),
    )(parts, cnt, x, Wl, b, Wr)


def kernel(x, edge_index, W1l, b1l, W1r, W2l, b2l, W2r):
    src = edge_index[0]
    dst = edge_index[1]
    pad = E_PAD - E
    src3 = jnp.concatenate(
        [src, jnp.zeros((pad,), jnp.int32)]).reshape(NW, NB, B)
    dst3 = jnp.concatenate(
        [dst, jnp.full((pad,), N, jnp.int32)]).reshape(NW, NB, B)
    zeros = jnp.zeros((RPS, 128), jnp.float32)
    ones = jnp.ones((B, 128), jnp.float32)

    cntp = _counts(ones, dst3, zeros)
    cnt = cntp[:, :, 0:1]

    p0 = _agg(x[:, :128], src3, dst3, zeros)
    p1 = _agg(x[:, 128:], src3, dst3, zeros)
    parts1 = jnp.concatenate([p0, p1], axis=2)

    x_pad = jnp.pad(x, ((0, N_ACC - N), (0, 0)))
    h = _dense(parts1, cnt, x_pad, W1l, b1l.reshape(1, -1), W1r, relu=True)

    p2 = [_agg(h[:, k * 128:(k + 1) * 128], src3, dst3, zeros)
          for k in range(4)]
    parts2 = jnp.concatenate(p2, axis=2)
    out = _dense(parts2, cnt, h, W2l, b2l.reshape(1, -1), W2r, relu=False)
    return out[:N]
